# baseline (speedup 1.0000x reference)
import jax
import jax.numpy as jnp
from jax import lax
from jax.experimental import pallas as pl
from jax.experimental.pallas import tpu as pltpu
from jax.experimental.pallas import tpu_sc as plsc

N = 50000
NODES = 50176
E = 800000
E_PAD = 819200
H = 64
HH = H // 2
OUT = 128
G = 512
PG = 544
CHUNK = 1024
NC = 2
NS = 16
LANES = 16

_mesh = plsc.VectorSubcoreMesh(core_axis_name="c", subcore_axis_name="s")
_sc_params = pltpu.CompilerParams(use_tc_tiling_on_sc=False)
_sc_params_nl = pltpu.CompilerParams(use_tc_tiling_on_sc=False,
                                     needs_layout_passes=False)


def _zero_fill(ref, rows, cols):
    zv = jnp.zeros((LANES,), jnp.float32)

    @pl.loop(0, rows)
    def _(r):
        @pl.loop(0, cols // LANES)
        def _(cc):
            ref[r, pl.ds(cc * LANES, LANES)] = zv


CHUNK_W = 400


def _agg64_body(tbl_hbm, src_hbm, dst_hbm, out_hbm,
                idx_g0, idx_g1, idx_d0, idx_d1, rows_i0, rows_i1, rows_f,
                gsem0, gsem1, acc):
    c = lax.axis_index("c")
    s = lax.axis_index("s")

    _zero_fill(rows_f, 392, HH)
    row0 = s * (NODES // NS)

    @pl.loop(0, 8)
    def _(k):
        pltpu.sync_copy(rows_f.at[pl.ds(0, 392)],
                        acc.at[pl.ds(row0 + k * 392, 392)])

    plsc.subcore_barrier()

    ebase = s * (E_PAD // NS)
    mask_hi = jnp.int32(-65536)

    def _load_idx(base, idx_g, idx_d):
        pltpu.sync_copy(src_hbm.at[pl.ds(base, CHUNK_W)], idx_g)
        pltpu.sync_copy(dst_hbm.at[pl.ds(base, CHUNK_W)], idx_d)

        @pl.loop(0, CHUNK_W // LANES)
        def _(i):
            v = idx_g[pl.ds(i * LANES, LANES)]
            idx_g[pl.ds(i * LANES, LANES)] = v * 2 + c

    def _convert_scatter(rows_i, idx_d):
        @pl.loop(0, CHUNK_W)
        def _(r):
            v = rows_i[r, pl.ds(0, LANES)]
            rows_f[r, pl.ds(0, LANES)] = plsc.bitcast(
                lax.shift_left(v, 16), jnp.float32)
            rows_f[r, pl.ds(LANES, LANES)] = plsc.bitcast(
                lax.bitwise_and(v, mask_hi), jnp.float32)

        pltpu.sync_copy(rows_f, acc.at[idx_d], add=True)

    @pl.loop(0, E_PAD // NS // CHUNK_W // 2)
    def _(m):
        base = ebase + m * (2 * CHUNK_W)
        _load_idx(base, idx_g0, idx_d0)
        g0 = pltpu.async_copy(tbl_hbm.at[idx_g0], rows_i0, gsem0)
        _load_idx(base + CHUNK_W, idx_g1, idx_d1)
        g1 = pltpu.async_copy(tbl_hbm.at[idx_g1], rows_i1, gsem1)
        g0.wait()
        _convert_scatter(rows_i0, idx_d0)
        g1.wait()
        _convert_scatter(rows_i1, idx_d1)

    plsc.subcore_barrier()
    pltpu.sync_copy(acc.at[pl.ds(row0, NODES // NS)],
                    out_hbm.at[pl.ds(c * NODES + row0, NODES // NS)])


def _run_agg64(h_bf16, src_pad, dst_pad):
    tbl = lax.bitcast_convert_type(
        h_bf16.reshape(2 * NODES, LANES, 2), jnp.int32)
    kern = pl.kernel(
        _agg64_body,
        out_type=jax.ShapeDtypeStruct((2 * NODES, HH), jnp.float32),
        mesh=_mesh,
        compiler_params=_sc_params_nl,
        scratch_types=[
            pltpu.VMEM((CHUNK_W,), jnp.int32),
            pltpu.VMEM((CHUNK_W,), jnp.int32),
            pltpu.VMEM((CHUNK_W,), jnp.int32),
            pltpu.VMEM((CHUNK_W,), jnp.int32),
            pltpu.VMEM((CHUNK_W, LANES), jnp.int32),
            pltpu.VMEM((CHUNK_W, LANES), jnp.int32),
            pltpu.VMEM((CHUNK_W, HH), jnp.float32),
            pltpu.SemaphoreType.DMA,
            pltpu.SemaphoreType.DMA,
            pltpu.VMEM_SHARED((NODES, HH), jnp.float32),
        ],
    )
    return kern(tbl, src_pad, dst_pad)


CHUNK_1 = 800


def _agg1_body(x16_hbm, src_hbm, dst_hbm, out_hbm,
               idx_s0, idx_s1, idx_d0, idx_d1, rows0, rows1,
               gsem0, gsem1, ssem0, ssem1, acc):
    c = lax.axis_index("c")
    s = lax.axis_index("s")

    _zero_fill(rows0, 392, 16)
    row0 = s * (NODES // NS)

    @pl.loop(0, 8)
    def _(k):
        pltpu.sync_copy(rows0.at[pl.ds(0, 392)],
                        acc.at[pl.ds(row0 + k * 392, 392)])

    plsc.subcore_barrier()

    wid = c * NS + s
    ebase = wid * (E_PAD // (NC * NS))

    @pl.loop(0, E_PAD // (NC * NS) // CHUNK_1 // 2)
    def _(m):
        base = ebase + m * (2 * CHUNK_1)
        pltpu.sync_copy(src_hbm.at[pl.ds(base, CHUNK_1)], idx_s0)
        pltpu.sync_copy(dst_hbm.at[pl.ds(base, CHUNK_1)], idx_d0)
        g0 = pltpu.async_copy(x16_hbm.at[idx_s0], rows0, gsem0)
        pltpu.sync_copy(src_hbm.at[pl.ds(base + CHUNK_1, CHUNK_1)], idx_s1)
        pltpu.sync_copy(dst_hbm.at[pl.ds(base + CHUNK_1, CHUNK_1)], idx_d1)
        g1 = pltpu.async_copy(x16_hbm.at[idx_s1], rows1, gsem1)
        g0.wait()
        s0 = pltpu.async_copy(rows0, acc.at[idx_d0], ssem0, add=True)
        g1.wait()
        s1 = pltpu.async_copy(rows1, acc.at[idx_d1], ssem1, add=True)
        s0.wait()
        s1.wait()

    plsc.subcore_barrier()
    pltpu.sync_copy(acc.at[pl.ds(row0, NODES // NS)],
                    out_hbm.at[pl.ds(c * NODES + row0, NODES // NS)])


def _run_agg1(x16, src_pad, dst_pad):
    kern = pl.kernel(
        _agg1_body,
        out_type=jax.ShapeDtypeStruct((2 * NODES, 16), jnp.float32),
        mesh=_mesh,
        compiler_params=_sc_params,
        scratch_types=[
            pltpu.VMEM((CHUNK_1,), jnp.int32),
            pltpu.VMEM((CHUNK_1,), jnp.int32),
            pltpu.VMEM((CHUNK_1,), jnp.int32),
            pltpu.VMEM((CHUNK_1,), jnp.int32),
            pltpu.VMEM((CHUNK_1, 16), jnp.float32),
            pltpu.VMEM((CHUNK_1, 16), jnp.float32),
            pltpu.SemaphoreType.DMA,
            pltpu.SemaphoreType.DMA,
            pltpu.SemaphoreType.DMA,
            pltpu.SemaphoreType.DMA,
            pltpu.VMEM_SHARED((NODES, 16), jnp.float32),
        ],
    )
    return kern(x16, src_pad, dst_pad)


def _pool_body(h_hbm, batch_hbm, out_hbm, idx_b, rows_v, zbuf, acc):
    c = lax.axis_index("c")
    s = lax.axis_index("s")

    _zero_fill(zbuf, PG // NS, H)
    row0 = s * (PG // NS)
    pltpu.sync_copy(zbuf, acc.at[pl.ds(row0, PG // NS)])
    plsc.subcore_barrier()

    wid = c * NS + s
    nbase = wid * (NODES // (NC * NS))

    @pl.loop(0, 2)
    def _(ch):
        base = nbase + ch * 784
        pltpu.sync_copy(batch_hbm.at[pl.ds(base, 784)], idx_b)
        pltpu.sync_copy(h_hbm.at[pl.ds(base, 784)], rows_v)
        pltpu.sync_copy(rows_v, acc.at[idx_b], add=True)

    plsc.subcore_barrier()
    pltpu.sync_copy(acc.at[pl.ds(row0, PG // NS)],
                    out_hbm.at[pl.ds(c * PG + row0, PG // NS)])


def _run_pool(h, batch_pad):
    kern = pl.kernel(
        _pool_body,
        out_type=jax.ShapeDtypeStruct((2 * PG, H), jnp.float32),
        mesh=_mesh,
        compiler_params=_sc_params,
        scratch_types=[
            pltpu.VMEM((784,), jnp.int32),
            pltpu.VMEM((784, H), jnp.float32),
            pltpu.VMEM((PG // NS, H), jnp.float32),
            pltpu.VMEM_SHARED((PG, H), jnp.float32),
        ],
    )
    return kern(h, batch_pad)


_BLK = 1024
_NBLK = NODES // _BLK


def _mlp1_tc(x_ref, a0_ref, a1_ref, w1_ref, b1_ref, w2_ref, b2_ref,
             o_ref, ob_ref):
    s = x_ref[...] + a0_ref[:, :1] + a1_ref[:, :1]
    h = jnp.maximum(s * w1_ref[...] + b1_ref[...], 0.0)
    h = jnp.dot(h, w2_ref[...], preferred_element_type=jnp.float32)
    h = jnp.maximum(h + b2_ref[...], 0.0)
    o_ref[...] = h
    ob_ref[...] = h.astype(jnp.bfloat16)


def _run_mlp1(x_pad, agg16, W1, b1, W2, b2):
    return pl.pallas_call(
        _mlp1_tc,
        grid=(_NBLK,),
        in_specs=[
            pl.BlockSpec((_BLK, 1), lambda i: (i, 0)),
            pl.BlockSpec((_BLK, 16), lambda i: (i, 0)),
            pl.BlockSpec((_BLK, 16), lambda i: (i + _NBLK, 0)),
            pl.BlockSpec((1, H), lambda i: (0, 0)),
            pl.BlockSpec((1, H), lambda i: (0, 0)),
            pl.BlockSpec((H, H), lambda i: (0, 0)),
            pl.BlockSpec((1, H), lambda i: (0, 0)),
        ],
        out_specs=[pl.BlockSpec((_BLK, H), lambda i: (i, 0)),
                   pl.BlockSpec((_BLK, H), lambda i: (i, 0))],
        out_shape=[jax.ShapeDtypeStruct((NODES, H), jnp.float32),
                   jax.ShapeDtypeStruct((NODES, H), jnp.bfloat16)],
    )(x_pad, agg16, agg16, W1, b1.reshape(1, H), W2, b2.reshape(1, H))


def _mlp23_tc(h_ref, a0_ref, a1_ref, w1c_ref, b1_ref, w2_ref, b2_ref,
              o_ref, ob_ref):
    hin = jnp.concatenate([h_ref[...], a0_ref[...], a1_ref[...]], axis=1)
    h = jnp.dot(hin, w1c_ref[...], preferred_element_type=jnp.float32)
    h = jnp.maximum(h + b1_ref[...], 0.0)
    h = jnp.dot(h, w2_ref[...], preferred_element_type=jnp.float32)
    h = jnp.maximum(h + b2_ref[...], 0.0)
    o_ref[...] = h
    ob_ref[...] = h.astype(jnp.bfloat16)


def _run_mlp23(h_prev, agg, W1cat, b1, W2, b2):
    return pl.pallas_call(
        _mlp23_tc,
        grid=(_NBLK,),
        in_specs=[
            pl.BlockSpec((_BLK, H), lambda i: (i, 0)),
            pl.BlockSpec((_BLK, HH), lambda i: (i, 0)),
            pl.BlockSpec((_BLK, HH), lambda i: (i + _NBLK, 0)),
            pl.BlockSpec((2 * H, H), lambda i: (0, 0)),
            pl.BlockSpec((1, H), lambda i: (0, 0)),
            pl.BlockSpec((H, H), lambda i: (0, 0)),
            pl.BlockSpec((1, H), lambda i: (0, 0)),
        ],
        out_specs=[pl.BlockSpec((_BLK, H), lambda i: (i, 0)),
                   pl.BlockSpec((_BLK, H), lambda i: (i, 0))],
        out_shape=[jax.ShapeDtypeStruct((NODES, H), jnp.float32),
                   jax.ShapeDtypeStruct((NODES, H), jnp.bfloat16)],
    )(h_prev, agg, agg, W1cat, b1.reshape(1, H), W2, b2.reshape(1, H))


_AGG_COL_TRUE = [hc * HH + (2 * i if i < LANES else 2 * (i - LANES) + 1)
                 for hc in range(2) for i in range(HH)]


def _w1cat(W1):
    return jnp.concatenate([W1, W1[jnp.array(_AGG_COL_TRUE), :]], axis=0)


def _fc_tc(p0_ref, p1_ref, w_ref, b_ref, o_ref):
    p = p0_ref[:G, :] + p1_ref[:G, :]
    o_ref[...] = jnp.dot(p, w_ref[...],
                         preferred_element_type=jnp.float32) + b_ref[...]


def _run_fc(pacc, fcW, fcb):
    return pl.pallas_call(
        _fc_tc,
        grid=(1,),
        in_specs=[
            pl.BlockSpec((PG, H), lambda i: (0, 0)),
            pl.BlockSpec((PG, H), lambda i: (1, 0)),
            pl.BlockSpec((H, OUT), lambda i: (0, 0)),
            pl.BlockSpec((1, OUT), lambda i: (0, 0)),
        ],
        out_specs=pl.BlockSpec((G, OUT), lambda i: (0, 0)),
        out_shape=jax.ShapeDtypeStruct((G, OUT), jnp.float32),
    )(pacc, pacc, fcW, fcb.reshape(1, OUT))


def _forward(x, edge_index, batch, params):
    (c1W1, c1b1, c1W2, c1b2, c2W1, c2b1, c2W2, c2b2,
     c3W1, c3b1, c3W2, c3b2, fcW, fcb) = params

    src_pad = jnp.concatenate(
        [edge_index[0], jnp.zeros((E_PAD - E,), jnp.int32)])
    dst_pad = jnp.concatenate(
        [edge_index[1], jnp.full((E_PAD - E,), NODES - 1, jnp.int32)])
    batch_pad = jnp.concatenate([batch, jnp.full((NODES - N,), G, jnp.int32)])
    x_pad = jnp.pad(x, ((0, NODES - N), (0, 0)))
    x16 = jnp.pad(x, ((0, NODES - N), (0, 15)))

    agg16 = _run_agg1(x16, src_pad, dst_pad)
    h1, h1b = _run_mlp1(x_pad, agg16, c1W1, c1b1, c1W2, c1b2)
    agg2 = _run_agg64(h1b, src_pad, dst_pad)
    h2, h2b = _run_mlp23(h1, agg2, _w1cat(c2W1), c2b1, c2W2, c2b2)
    agg3 = _run_agg64(h2b, src_pad, dst_pad)
    h3, _ = _run_mlp23(h2, agg3, _w1cat(c3W1), c3b1, c3W2, c3b2)
    pacc = _run_pool(h3, batch_pad)
    return _run_fc(pacc, fcW, fcb)


def kernel(x_a, edge_index_a, batch_a, x_p, edge_index_p, batch_p,
           x_n, edge_index_n, batch_n,
           c1W1, c1b1, c1W2, c1b2, c2W1, c2b1, c2W2, c2b2,
           c3W1, c3b1, c3W2, c3b2, fcW, fcb):
    params = (c1W1, c1b1, c1W2, c1b2, c2W1, c2b1, c2W2, c2b2,
              c3W1, c3b1, c3W2, c3b2, fcW, fcb)
    a = _forward(x_a, edge_index_a, batch_a, params)
    p = _forward(x_p, edge_index_p, batch_p, params)
    n = _forward(x_n, edge_index_n, batch_n, params)
    return (a, p, n)


# speedup vs baseline: 3.7798x; 3.7798x over previous
import jax
import jax.numpy as jnp
from jax import lax
from jax.experimental import pallas as pl
from jax.experimental.pallas import tpu as pltpu
from jax.experimental.pallas import tpu_sc as plsc

N = 50000
NODES = 50176
E = 800000
E_PAD = 819200
H = 64
HH = H // 2
OUT = 128
G = 512
PG = 544
CHUNK = 1024
NC = 2
NS = 16
LANES = 16

_mesh = plsc.VectorSubcoreMesh(core_axis_name="c", subcore_axis_name="s")
_sc_params = pltpu.CompilerParams(use_tc_tiling_on_sc=False)
_sc_params_nl = pltpu.CompilerParams(use_tc_tiling_on_sc=False,
                                     needs_layout_passes=False)


def _zero_fill(ref, rows, cols):
    zv = jnp.zeros((LANES,), jnp.float32)

    @pl.loop(0, rows)
    def _(r):
        @pl.loop(0, cols // LANES)
        def _(cc):
            ref[r, pl.ds(cc * LANES, LANES)] = zv


CHUNK_W = 400


def _agg64_body(h_flat, src_hbm, dst_hbm, out_hbm,
                idx_g0, idx_g1, idx_d0, idx_d1, rows0, rows1,
                gsem0, gsem1, ssem0, ssem1, acc):
    c = lax.axis_index("c")
    s = lax.axis_index("s")

    _zero_fill(rows0, 392, HH)
    row0 = s * (NODES // NS)

    @pl.loop(0, 8)
    def _(k):
        pltpu.sync_copy(rows0.at[pl.ds(0, 392)],
                        acc.at[pl.ds(row0 + k * 392, 392)])

    plsc.subcore_barrier()

    ebase = s * (E_PAD // NS)

    def _load_idx(base, idx_g, idx_d):
        pltpu.sync_copy(src_hbm.at[pl.ds(base, CHUNK_W)], idx_g)
        pltpu.sync_copy(dst_hbm.at[pl.ds(base, CHUNK_W)], idx_d)

        @pl.loop(0, CHUNK_W // LANES)
        def _(i):
            v = idx_g[pl.ds(i * LANES, LANES)]
            idx_g[pl.ds(i * LANES, LANES)] = v * 2 + c

    @pl.loop(0, E_PAD // NS // CHUNK_W // 2)
    def _(m):
        base = ebase + m * (2 * CHUNK_W)
        _load_idx(base, idx_g0, idx_d0)
        g0 = pltpu.async_copy(h_flat.at[idx_g0], rows0, gsem0)
        _load_idx(base + CHUNK_W, idx_g1, idx_d1)
        g1 = pltpu.async_copy(h_flat.at[idx_g1], rows1, gsem1)
        g0.wait()
        s0 = pltpu.async_copy(rows0, acc.at[idx_d0], ssem0, add=True)
        g1.wait()
        s1 = pltpu.async_copy(rows1, acc.at[idx_d1], ssem1, add=True)
        s0.wait()
        s1.wait()

    plsc.subcore_barrier()
    pltpu.sync_copy(acc.at[pl.ds(row0, NODES // NS)],
                    out_hbm.at[pl.ds(c * NODES + row0, NODES // NS)])


def _run_agg64(h, src_pad, dst_pad):
    h_flat = h.reshape(2 * NODES, HH)
    kern = pl.kernel(
        _agg64_body,
        out_type=jax.ShapeDtypeStruct((2 * NODES, HH), jnp.float32),
        mesh=_mesh,
        compiler_params=_sc_params,
        scratch_types=[
            pltpu.VMEM((CHUNK_W,), jnp.int32),
            pltpu.VMEM((CHUNK_W,), jnp.int32),
            pltpu.VMEM((CHUNK_W,), jnp.int32),
            pltpu.VMEM((CHUNK_W,), jnp.int32),
            pltpu.VMEM((CHUNK_W, HH), jnp.float32),
            pltpu.VMEM((CHUNK_W, HH), jnp.float32),
            pltpu.SemaphoreType.DMA,
            pltpu.SemaphoreType.DMA,
            pltpu.SemaphoreType.DMA,
            pltpu.SemaphoreType.DMA,
            pltpu.VMEM_SHARED((NODES, HH), jnp.float32),
        ],
    )
    return kern(h_flat, src_pad, dst_pad)


CHUNK_1 = 800


def _agg1_body(x16_hbm, src_hbm, dst_hbm, out_hbm,
               idx_s0, idx_s1, idx_d0, idx_d1, rows0, rows1,
               gsem0, gsem1, ssem0, ssem1, acc):
    c = lax.axis_index("c")
    s = lax.axis_index("s")

    _zero_fill(rows0, 392, 16)
    row0 = s * (NODES // NS)

    @pl.loop(0, 8)
    def _(k):
        pltpu.sync_copy(rows0.at[pl.ds(0, 392)],
                        acc.at[pl.ds(row0 + k * 392, 392)])

    plsc.subcore_barrier()

    wid = c * NS + s
    ebase = wid * (E_PAD // (NC * NS))

    @pl.loop(0, E_PAD // (NC * NS) // CHUNK_1 // 2)
    def _(m):
        base = ebase + m * (2 * CHUNK_1)
        pltpu.sync_copy(src_hbm.at[pl.ds(base, CHUNK_1)], idx_s0)
        pltpu.sync_copy(dst_hbm.at[pl.ds(base, CHUNK_1)], idx_d0)
        g0 = pltpu.async_copy(x16_hbm.at[idx_s0], rows0, gsem0)
        pltpu.sync_copy(src_hbm.at[pl.ds(base + CHUNK_1, CHUNK_1)], idx_s1)
        pltpu.sync_copy(dst_hbm.at[pl.ds(base + CHUNK_1, CHUNK_1)], idx_d1)
        g1 = pltpu.async_copy(x16_hbm.at[idx_s1], rows1, gsem1)
        g0.wait()
        s0 = pltpu.async_copy(rows0, acc.at[idx_d0], ssem0, add=True)
        g1.wait()
        s1 = pltpu.async_copy(rows1, acc.at[idx_d1], ssem1, add=True)
        s0.wait()
        s1.wait()

    plsc.subcore_barrier()
    pltpu.sync_copy(acc.at[pl.ds(row0, NODES // NS)],
                    out_hbm.at[pl.ds(c * NODES + row0, NODES // NS)])


def _run_agg1(x16, src_pad, dst_pad):
    kern = pl.kernel(
        _agg1_body,
        out_type=jax.ShapeDtypeStruct((2 * NODES, 16), jnp.float32),
        mesh=_mesh,
        compiler_params=_sc_params,
        scratch_types=[
            pltpu.VMEM((CHUNK_1,), jnp.int32),
            pltpu.VMEM((CHUNK_1,), jnp.int32),
            pltpu.VMEM((CHUNK_1,), jnp.int32),
            pltpu.VMEM((CHUNK_1,), jnp.int32),
            pltpu.VMEM((CHUNK_1, 16), jnp.float32),
            pltpu.VMEM((CHUNK_1, 16), jnp.float32),
            pltpu.SemaphoreType.DMA,
            pltpu.SemaphoreType.DMA,
            pltpu.SemaphoreType.DMA,
            pltpu.SemaphoreType.DMA,
            pltpu.VMEM_SHARED((NODES, 16), jnp.float32),
        ],
    )
    return kern(x16, src_pad, dst_pad)


def _pool_body(h_hbm, batch_hbm, out_hbm, idx_b, rows_v, zbuf, acc):
    c = lax.axis_index("c")
    s = lax.axis_index("s")

    _zero_fill(zbuf, PG // NS, H)
    row0 = s * (PG // NS)
    pltpu.sync_copy(zbuf, acc.at[pl.ds(row0, PG // NS)])
    plsc.subcore_barrier()

    wid = c * NS + s
    nbase = wid * (NODES // (NC * NS))

    @pl.loop(0, 2)
    def _(ch):
        base = nbase + ch * 784
        pltpu.sync_copy(batch_hbm.at[pl.ds(base, 784)], idx_b)
        pltpu.sync_copy(h_hbm.at[pl.ds(base, 784)], rows_v)
        pltpu.sync_copy(rows_v, acc.at[idx_b], add=True)

    plsc.subcore_barrier()
    pltpu.sync_copy(acc.at[pl.ds(row0, PG // NS)],
                    out_hbm.at[pl.ds(c * PG + row0, PG // NS)])


def _run_pool(h, batch_pad):
    kern = pl.kernel(
        _pool_body,
        out_type=jax.ShapeDtypeStruct((2 * PG, H), jnp.float32),
        mesh=_mesh,
        compiler_params=_sc_params,
        scratch_types=[
            pltpu.VMEM((784,), jnp.int32),
            pltpu.VMEM((784, H), jnp.float32),
            pltpu.VMEM((PG // NS, H), jnp.float32),
            pltpu.VMEM_SHARED((PG, H), jnp.float32),
        ],
    )
    return kern(h, batch_pad)


_BLK = 1024
_NBLK = NODES // _BLK


def _mlp1_tc(x_ref, a0_ref, a1_ref, w1_ref, b1_ref, w2_ref, b2_ref, o_ref):
    s = x_ref[...] + a0_ref[:, :1] + a1_ref[:, :1]
    h = jnp.maximum(s * w1_ref[...] + b1_ref[...], 0.0)
    h = jnp.dot(h, w2_ref[...], preferred_element_type=jnp.float32)
    o_ref[...] = jnp.maximum(h + b2_ref[...], 0.0)


def _run_mlp1(x_pad, agg16, W1, b1, W2, b2):
    return pl.pallas_call(
        _mlp1_tc,
        grid=(_NBLK,),
        in_specs=[
            pl.BlockSpec((_BLK, 1), lambda i: (i, 0)),
            pl.BlockSpec((_BLK, 16), lambda i: (i, 0)),
            pl.BlockSpec((_BLK, 16), lambda i: (i + _NBLK, 0)),
            pl.BlockSpec((1, H), lambda i: (0, 0)),
            pl.BlockSpec((1, H), lambda i: (0, 0)),
            pl.BlockSpec((H, H), lambda i: (0, 0)),
            pl.BlockSpec((1, H), lambda i: (0, 0)),
        ],
        out_specs=pl.BlockSpec((_BLK, H), lambda i: (i, 0)),
        out_shape=jax.ShapeDtypeStruct((NODES, H), jnp.float32),
    )(x_pad, agg16, agg16, W1, b1.reshape(1, H), W2, b2.reshape(1, H))


def _mlp23_tc(h_ref, a0_ref, a1_ref, w1c_ref, b1_ref, w2_ref, b2_ref, o_ref):
    hin = jnp.concatenate([h_ref[...], a0_ref[...], a1_ref[...]], axis=1)
    h = jnp.dot(hin, w1c_ref[...], preferred_element_type=jnp.float32)
    h = jnp.maximum(h + b1_ref[...], 0.0)
    h = jnp.dot(h, w2_ref[...], preferred_element_type=jnp.float32)
    o_ref[...] = jnp.maximum(h + b2_ref[...], 0.0)


def _run_mlp23(h_prev, agg, W1cat, b1, W2, b2):
    return pl.pallas_call(
        _mlp23_tc,
        grid=(_NBLK,),
        in_specs=[
            pl.BlockSpec((_BLK, H), lambda i: (i, 0)),
            pl.BlockSpec((_BLK, HH), lambda i: (i, 0)),
            pl.BlockSpec((_BLK, HH), lambda i: (i + _NBLK, 0)),
            pl.BlockSpec((2 * H, H), lambda i: (0, 0)),
            pl.BlockSpec((1, H), lambda i: (0, 0)),
            pl.BlockSpec((H, H), lambda i: (0, 0)),
            pl.BlockSpec((1, H), lambda i: (0, 0)),
        ],
        out_specs=pl.BlockSpec((_BLK, H), lambda i: (i, 0)),
        out_shape=jax.ShapeDtypeStruct((NODES, H), jnp.float32),
    )(h_prev, agg, agg, W1cat, b1.reshape(1, H), W2, b2.reshape(1, H))


def _w1cat(W1):
    return jnp.concatenate([W1, W1], axis=0)


def _fc_tc(p0_ref, p1_ref, w_ref, b_ref, o_ref):
    p = p0_ref[:G, :] + p1_ref[:G, :]
    o_ref[...] = jnp.dot(p, w_ref[...],
                         preferred_element_type=jnp.float32) + b_ref[...]


def _run_fc(pacc, fcW, fcb):
    return pl.pallas_call(
        _fc_tc,
        grid=(1,),
        in_specs=[
            pl.BlockSpec((PG, H), lambda i: (0, 0)),
            pl.BlockSpec((PG, H), lambda i: (1, 0)),
            pl.BlockSpec((H, OUT), lambda i: (0, 0)),
            pl.BlockSpec((1, OUT), lambda i: (0, 0)),
        ],
        out_specs=pl.BlockSpec((G, OUT), lambda i: (0, 0)),
        out_shape=jax.ShapeDtypeStruct((G, OUT), jnp.float32),
    )(pacc, pacc, fcW, fcb.reshape(1, OUT))


def _forward(x, edge_index, batch, params):
    (c1W1, c1b1, c1W2, c1b2, c2W1, c2b1, c2W2, c2b2,
     c3W1, c3b1, c3W2, c3b2, fcW, fcb) = params

    src_pad = jnp.concatenate(
        [edge_index[0], jnp.zeros((E_PAD - E,), jnp.int32)])
    dst_pad = jnp.concatenate(
        [edge_index[1], jnp.full((E_PAD - E,), NODES - 1, jnp.int32)])
    batch_pad = jnp.concatenate([batch, jnp.full((NODES - N,), G, jnp.int32)])
    x_pad = jnp.pad(x, ((0, NODES - N), (0, 0)))
    x16 = jnp.pad(x, ((0, NODES - N), (0, 15)))

    agg16 = _run_agg1(x16, src_pad, dst_pad)
    h1 = _run_mlp1(x_pad, agg16, c1W1, c1b1, c1W2, c1b2)
    agg2 = _run_agg64(h1, src_pad, dst_pad)
    h2 = _run_mlp23(h1, agg2, _w1cat(c2W1), c2b1, c2W2, c2b2)
    agg3 = _run_agg64(h2, src_pad, dst_pad)
    h3 = _run_mlp23(h2, agg3, _w1cat(c3W1), c3b1, c3W2, c3b2)
    pacc = _run_pool(h3, batch_pad)
    return _run_fc(pacc, fcW, fcb)


def kernel(x_a, edge_index_a, batch_a, x_p, edge_index_p, batch_p,
           x_n, edge_index_n, batch_n,
           c1W1, c1b1, c1W2, c1b2, c2W1, c2b1, c2W2, c2b2,
           c3W1, c3b1, c3W2, c3b2, fcW, fcb):
    params = (c1W1, c1b1, c1W2, c1b2, c2W1, c2b1, c2W2, c2b2,
              c3W1, c3b1, c3W2, c3b2, fcW, fcb)
    a = _forward(x_a, edge_index_a, batch_a, params)
    p = _forward(x_p, edge_index_p, batch_p, params)
    n = _forward(x_n, edge_index_n, batch_n, params)
    return (a, p, n)
